# Initial kernel scaffold; baseline (speedup 1.0000x reference)
import jax
import jax.numpy as jnp
from jax.experimental import pallas as pl


def kernel(A1, Q1, A2, Q2, S, W1, b1, W2, b2, bn_gamma, bn_beta, bn_mean, bn_var, pw_w, dw_w, dw_b, fc_w, fc_b):
    raise NotImplementedError("write your pallas kernel here")


# speedup vs baseline: 1.0057x; 1.0057x over previous
import jax
import jax.numpy as jnp
from jax.experimental import pallas as pl

N = 4096
F = 128
HH = 128
WW = 128
NPIX = HH * WW


def _relu_mm_kernel(x_ref, w_ref, b_ref, o_ref):
    acc = jnp.dot(x_ref[...], w_ref[...], preferred_element_type=jnp.float32)
    o_ref[...] = jnp.maximum(acc + b_ref[...], 0.0)


def _input_transform(Q, Wcat, bcat):
    BM = 512
    return pl.pallas_call(
        _relu_mm_kernel,
        grid=(N // BM,),
        in_specs=[
            pl.BlockSpec((BM, F), lambda i: (i, 0)),
            pl.BlockSpec((F, 192), lambda i: (0, 0)),
            pl.BlockSpec((1, 192), lambda i: (0, 0)),
        ],
        out_specs=pl.BlockSpec((BM, 192), lambda i: (i, 0)),
        out_shape=jax.ShapeDtypeStruct((N, 192), jnp.float32),
    )(Q, Wcat, bcat)


def _apass_kernel(a_ref, x_ref, o_ref):
    o_ref[...] = jnp.dot(a_ref[...], x_ref[...], preferred_element_type=jnp.float32)


def _a_apply(A, X):
    BM = 256
    W = X.shape[1]
    return pl.pallas_call(
        _apass_kernel,
        grid=(N // BM,),
        in_specs=[
            pl.BlockSpec((BM, N), lambda i: (i, 0)),
            pl.BlockSpec((N, W), lambda i: (0, 0)),
        ],
        out_specs=pl.BlockSpec((BM, W), lambda i: (i, 0)),
        out_shape=jax.ShapeDtypeStruct((N, W), jnp.float32),
    )(A, X)


def _amp_kernel(f1_ref, f2_ref, o_ref):
    f1 = f1_ref[...]
    f2 = f2_ref[...]
    n1 = jnp.maximum(jnp.sqrt(jnp.sum(f1 * f1, axis=0)), 1e-8)
    n2 = jnp.maximum(jnp.sqrt(jnp.sum(f2 * f2, axis=0)), 1e-8)
    cs = jnp.sum(f1 * f2, axis=0) / (n1 * n2)
    o_ref[...] = jax.nn.sigmoid(1.0 - cs)[None, :]


def _amp(f11, f21):
    return pl.pallas_call(
        _amp_kernel,
        out_shape=jax.ShapeDtypeStruct((1, 192), jnp.float32),
    )(f11, f21)


def _fg_kernel(f_ref, amp_ref, w_ref, o_ref):
    o_ref[...] = jnp.dot(f_ref[...] * amp_ref[...], w_ref[...],
                         preferred_element_type=jnp.float32)


def _fg(f, amp, W2cat):
    BM = 512
    return pl.pallas_call(
        _fg_kernel,
        grid=(N // BM,),
        in_specs=[
            pl.BlockSpec((BM, 192), lambda i: (i, 0)),
            pl.BlockSpec((1, 192), lambda i: (0, 0)),
            pl.BlockSpec((192, 192), lambda i: (0, 0)),
        ],
        out_specs=pl.BlockSpec((BM, 192), lambda i: (i, 0)),
        out_shape=jax.ShapeDtypeStruct((N, 192), jnp.float32),
    )(f, amp, W2cat)


def _mm_kernel(x_ref, w_ref, o_ref):
    o_ref[...] = jnp.dot(x_ref[...], w_ref[...], preferred_element_type=jnp.float32)


def _feats_project(feats0, Wp):
    BM = 512
    return pl.pallas_call(
        _mm_kernel,
        grid=(N // BM,),
        in_specs=[
            pl.BlockSpec((BM, 384), lambda i: (i, 0)),
            pl.BlockSpec((384, 32), lambda i: (0, 0)),
        ],
        out_specs=pl.BlockSpec((BM, 32), lambda i: (i, 0)),
        out_shape=jax.ShapeDtypeStruct((N, 32), jnp.float32),
    )(feats0, Wp)


def _s_kernel(s_ref, f_ref, c_ref, o_ref):
    y = jnp.dot(s_ref[...], f_ref[...], preferred_element_type=jnp.float32)
    y = y + c_ref[...]
    o_ref[...] = jnp.where(y >= 0, y, 0.01 * y)


def _s_matmul(S, F2, cp):
    BM = 512
    return pl.pallas_call(
        _s_kernel,
        grid=(NPIX // BM,),
        in_specs=[
            pl.BlockSpec((BM, N), lambda i: (i, 0)),
            pl.BlockSpec((N, 32), lambda i: (0, 0)),
            pl.BlockSpec((1, 32), lambda i: (0, 0)),
        ],
        out_specs=pl.BlockSpec((BM, 32), lambda i: (i, 0)),
        out_shape=jax.ShapeDtypeStruct((NPIX, 32), jnp.float32),
    )(S, F2, cp)


def _head_kernel(x_ref, dwk_ref, dwb_ref, fcw_ref, fcb_ref, o_ref):
    x = x_ref[...]
    zpad = jnp.zeros((129, 32), jnp.float32)
    xp = jnp.concatenate([zpad, x, zpad], axis=0)
    wcol = jax.lax.broadcasted_iota(jnp.int32, (NPIX, 1), 0) % WW
    acc = jnp.zeros((NPIX, 32), jnp.float32)
    k = 0
    for dy in (-1, 0, 1):
        for dx in (-1, 0, 1):
            s = dy * WW + dx
            sh = jax.lax.slice(xp, (129 + s, 0), (129 + s + NPIX, 32))
            if dx == -1:
                sh = jnp.where(wcol >= 1, sh, 0.0)
            elif dx == 1:
                sh = jnp.where(wcol <= WW - 2, sh, 0.0)
            acc = acc + sh * dwk_ref[k, :][None, :]
            k += 1
    y = acc + dwb_ref[...]
    y = jnp.where(y >= 0, y, 0.01 * y)
    logits = jnp.dot(y, fcw_ref[...], preferred_element_type=jnp.float32)
    logits = logits + fcb_ref[...]
    m = jnp.max(logits, axis=1, keepdims=True)
    e = jnp.exp(logits - m)
    o_ref[...] = e / jnp.sum(e, axis=1, keepdims=True)


def _head(X1, dwk, dwb, fcw, fcb):
    return pl.pallas_call(
        _head_kernel,
        out_shape=jax.ShapeDtypeStruct((NPIX, 16), jnp.float32),
    )(X1, dwk, dwb, fcw, fcb)


def kernel(A1, Q1, A2, Q2, S, W1, b1, W2, b2, bn_gamma, bn_beta, bn_mean,
           bn_var, pw_w, dw_w, dw_b, fc_w, fc_b):
    Wcat = jnp.concatenate([W1[0], W1[1], W1[2]], axis=1)
    bcat = jnp.reshape(b1, (1, 192))
    W2cat = jnp.concatenate([W2[0], W2[1], W2[2]], axis=1)

    def branch_sparse(A, Q):
        IN = _input_transform(Q, Wcat, bcat)
        Y1 = _a_apply(A, IN[:, 64:])
        s2 = _a_apply(A, Y1[:, 64:])
        return jnp.concatenate([IN[:, :64], Y1[:, :64], s2], axis=1)

    f11 = branch_sparse(A1, Q1)
    f21 = branch_sparse(A2, Q2)
    amp = _amp(f11, f21)

    def branch_dense(A, f):
        G = _fg(f, amp, W2cat)
        Y3 = _a_apply(A, G[:, 64:])
        d2 = _a_apply(A, Y3[:, 64:])
        return jnp.concatenate([G[:, :64], Y3[:, :64], d2], axis=1)

    f12 = branch_dense(A1, f11)
    f22 = branch_dense(A2, f21)
    feats0 = jnp.concatenate([f12, f22], axis=1)

    scale = bn_gamma / jnp.sqrt(bn_var + 1e-5)
    shift = bn_beta - bn_mean * scale
    pwT = pw_w[:, :, 0, 0].T
    Wp = scale[:, None] * pwT
    bvec = jnp.concatenate([jnp.reshape(b2, (192,))] * 2)[None, :]
    cp_total = bvec @ Wp + shift[None, :] @ pwT

    F2 = _feats_project(feats0, Wp)
    X1 = _s_matmul(S, F2, cp_total)

    dwk = jnp.transpose(dw_w[:, 0], (1, 2, 0)).reshape(9, 32)
    return _head(X1, dwk, dw_b[None, :], fc_w, fc_b[None, :])


# speedup vs baseline: 1.1220x; 1.1156x over previous
import jax
import jax.numpy as jnp
from jax.experimental import pallas as pl
from jax.experimental.pallas import tpu as pltpu

N = 4096
F = 128
HH = 128
WW = 128
NPIX = HH * WW


def _relu_mm_kernel(x_ref, w_ref, b_ref, o_ref):
    acc = jnp.dot(x_ref[...], w_ref[...], preferred_element_type=jnp.float32)
    o_ref[...] = jnp.maximum(acc + b_ref[...], 0.0)


def _input_transform(Q, Wcat, bcat):
    BM = 512
    return pl.pallas_call(
        _relu_mm_kernel,
        grid=(N // BM,),
        in_specs=[
            pl.BlockSpec((BM, F), lambda i: (i, 0)),
            pl.BlockSpec((F, 192), lambda i: (0, 0)),
            pl.BlockSpec((1, 192), lambda i: (0, 0)),
        ],
        out_specs=pl.BlockSpec((BM, 192), lambda i: (i, 0)),
        out_shape=jax.ShapeDtypeStruct((N, 192), jnp.float32),
    )(Q, Wcat, bcat)


_BM = 128
_NB = N // _BM


def _stage_kernel(a_ref, x_ref, y_ref, z_ref, abf_ref, xbf_ref):
    i = pl.program_id(0)

    @pl.when(i == 0)
    def _():
        xbf_ref[...] = x_ref[...].astype(jnp.bfloat16)

    @pl.when(i < _NB)
    def _():
        ab = a_ref[...].astype(jnp.bfloat16)
        abf_ref[pl.ds(i * _BM, _BM), :] = ab
        y_ref[pl.ds(i * _BM, _BM), :] = jnp.dot(
            ab, xbf_ref[...], preferred_element_type=jnp.float32)

    @pl.when(i == _NB)
    def _():
        t = y_ref[:, 64:128].astype(jnp.bfloat16)
        z_ref[...] = jnp.dot(abf_ref[...], t, preferred_element_type=jnp.float32)


def _stage(A, X):
    return pl.pallas_call(
        _stage_kernel,
        grid=(_NB + 1,),
        in_specs=[
            pl.BlockSpec((_BM, N), lambda i: (jnp.minimum(i, _NB - 1), 0)),
            pl.BlockSpec((N, 128), lambda i: (0, 0)),
        ],
        out_specs=[
            pl.BlockSpec((N, 128), lambda i: (0, 0)),
            pl.BlockSpec((N, 64), lambda i: (0, 0)),
        ],
        out_shape=[
            jax.ShapeDtypeStruct((N, 128), jnp.float32),
            jax.ShapeDtypeStruct((N, 64), jnp.float32),
        ],
        scratch_shapes=[
            pltpu.VMEM((N, N), jnp.bfloat16),
            pltpu.VMEM((N, 128), jnp.bfloat16),
        ],
    )(A, X)


def _amp_kernel(f1_ref, f2_ref, o_ref):
    f1 = f1_ref[...]
    f2 = f2_ref[...]
    n1 = jnp.maximum(jnp.sqrt(jnp.sum(f1 * f1, axis=0)), 1e-8)
    n2 = jnp.maximum(jnp.sqrt(jnp.sum(f2 * f2, axis=0)), 1e-8)
    cs = jnp.sum(f1 * f2, axis=0) / (n1 * n2)
    o_ref[...] = jax.nn.sigmoid(1.0 - cs)[None, :]


def _amp(f11, f21):
    return pl.pallas_call(
        _amp_kernel,
        out_shape=jax.ShapeDtypeStruct((1, 192), jnp.float32),
    )(f11, f21)


def _fg_kernel(f_ref, amp_ref, w_ref, o_ref):
    o_ref[...] = jnp.dot(f_ref[...] * amp_ref[...], w_ref[...],
                         preferred_element_type=jnp.float32)


def _fg(f, amp, W2cat):
    BM = 512
    return pl.pallas_call(
        _fg_kernel,
        grid=(N // BM,),
        in_specs=[
            pl.BlockSpec((BM, 192), lambda i: (i, 0)),
            pl.BlockSpec((1, 192), lambda i: (0, 0)),
            pl.BlockSpec((192, 192), lambda i: (0, 0)),
        ],
        out_specs=pl.BlockSpec((BM, 192), lambda i: (i, 0)),
        out_shape=jax.ShapeDtypeStruct((N, 192), jnp.float32),
    )(f, amp, W2cat)


def _mm_kernel(x_ref, w_ref, o_ref):
    o_ref[...] = jnp.dot(x_ref[...], w_ref[...], preferred_element_type=jnp.float32)


def _feats_project(feats0, Wp):
    BM = 512
    return pl.pallas_call(
        _mm_kernel,
        grid=(N // BM,),
        in_specs=[
            pl.BlockSpec((BM, 384), lambda i: (i, 0)),
            pl.BlockSpec((384, 32), lambda i: (0, 0)),
        ],
        out_specs=pl.BlockSpec((BM, 32), lambda i: (i, 0)),
        out_shape=jax.ShapeDtypeStruct((N, 32), jnp.float32),
    )(feats0, Wp)


def _s_kernel(s_ref, f_ref, c_ref, o_ref):
    y = jnp.dot(s_ref[...], f_ref[...], preferred_element_type=jnp.float32)
    y = y + c_ref[...]
    o_ref[...] = jnp.where(y >= 0, y, 0.01 * y)


def _s_matmul(S, F2, cp):
    BM = 512
    return pl.pallas_call(
        _s_kernel,
        grid=(NPIX // BM,),
        in_specs=[
            pl.BlockSpec((BM, N), lambda i: (i, 0)),
            pl.BlockSpec((N, 32), lambda i: (0, 0)),
            pl.BlockSpec((1, 32), lambda i: (0, 0)),
        ],
        out_specs=pl.BlockSpec((BM, 32), lambda i: (i, 0)),
        out_shape=jax.ShapeDtypeStruct((NPIX, 32), jnp.float32),
    )(S, F2, cp)


def _head_kernel(x_ref, dwk_ref, dwb_ref, fcw_ref, fcb_ref, o_ref):
    x = x_ref[...]
    zpad = jnp.zeros((129, 32), jnp.float32)
    xp = jnp.concatenate([zpad, x, zpad], axis=0)
    wcol = jax.lax.broadcasted_iota(jnp.int32, (NPIX, 1), 0) % WW
    acc = jnp.zeros((NPIX, 32), jnp.float32)
    k = 0
    for dy in (-1, 0, 1):
        for dx in (-1, 0, 1):
            s = dy * WW + dx
            sh = jax.lax.slice(xp, (129 + s, 0), (129 + s + NPIX, 32))
            if dx == -1:
                sh = jnp.where(wcol >= 1, sh, 0.0)
            elif dx == 1:
                sh = jnp.where(wcol <= WW - 2, sh, 0.0)
            acc = acc + sh * dwk_ref[k, :][None, :]
            k += 1
    y = acc + dwb_ref[...]
    y = jnp.where(y >= 0, y, 0.01 * y)
    logits = jnp.dot(y, fcw_ref[...], preferred_element_type=jnp.float32)
    logits = logits + fcb_ref[...]
    m = jnp.max(logits, axis=1, keepdims=True)
    e = jnp.exp(logits - m)
    o_ref[...] = e / jnp.sum(e, axis=1, keepdims=True)


def _head(X1, dwk, dwb, fcw, fcb):
    return pl.pallas_call(
        _head_kernel,
        out_shape=jax.ShapeDtypeStruct((NPIX, 16), jnp.float32),
    )(X1, dwk, dwb, fcw, fcb)


def kernel(A1, Q1, A2, Q2, S, W1, b1, W2, b2, bn_gamma, bn_beta, bn_mean,
           bn_var, pw_w, dw_w, dw_b, fc_w, fc_b):
    Wcat = jnp.concatenate([W1[0], W1[1], W1[2]], axis=1)
    bcat = jnp.reshape(b1, (1, 192))
    W2cat = jnp.concatenate([W2[0], W2[1], W2[2]], axis=1)

    def branch_sparse(A, Q):
        IN = _input_transform(Q, Wcat, bcat)
        Y1, s2 = _stage(A, IN[:, 64:])
        return jnp.concatenate([IN[:, :64], Y1[:, :64], s2], axis=1)

    f11 = branch_sparse(A1, Q1)
    f21 = branch_sparse(A2, Q2)
    amp = _amp(f11, f21)

    def branch_dense(A, f):
        G = _fg(f, amp, W2cat)
        Y3, d2 = _stage(A, G[:, 64:])
        return jnp.concatenate([G[:, :64], Y3[:, :64], d2], axis=1)

    f12 = branch_dense(A1, f11)
    f22 = branch_dense(A2, f21)
    feats0 = jnp.concatenate([f12, f22], axis=1)

    scale = bn_gamma / jnp.sqrt(bn_var + 1e-5)
    shift = bn_beta - bn_mean * scale
    pwT = pw_w[:, :, 0, 0].T
    Wp = scale[:, None] * pwT
    bvec = jnp.concatenate([jnp.reshape(b2, (192,))] * 2)[None, :]
    cp_total = bvec @ Wp + shift[None, :] @ pwT

    F2 = _feats_project(feats0, Wp)
    X1 = _s_matmul(S, F2, cp_total)

    dwk = jnp.transpose(dw_w[:, 0], (1, 2, 0)).reshape(9, 32)
    return _head(X1, dwk, dw_b[None, :], fc_w, fc_b[None, :])


# speedup vs baseline: 1.1786x; 1.0505x over previous
import jax
import jax.numpy as jnp
from jax.experimental import pallas as pl
from jax.experimental.pallas import tpu as pltpu

N = 4096
F = 128
HH = 128
WW = 128
NPIX = HH * WW


def _input_kernel(x_ref, w_ref, b_ref, s0_ref, h_ref):
    acc = jnp.dot(x_ref[...], w_ref[...], preferred_element_type=jnp.float32)
    acc = jnp.maximum(acc + b_ref[...], 0.0)
    s0_ref[...] = acc[:, :64]
    h_ref[...] = acc[:, 64:].astype(jnp.bfloat16)


def _input_transform(Q, Wcat, bcat):
    BM = 512
    return pl.pallas_call(
        _input_kernel,
        grid=(N // BM,),
        in_specs=[
            pl.BlockSpec((BM, F), lambda i: (i, 0)),
            pl.BlockSpec((F, 192), lambda i: (0, 0)),
            pl.BlockSpec((1, 192), lambda i: (0, 0)),
        ],
        out_specs=[
            pl.BlockSpec((BM, 64), lambda i: (i, 0)),
            pl.BlockSpec((BM, 128), lambda i: (i, 0)),
        ],
        out_shape=[
            jax.ShapeDtypeStruct((N, 64), jnp.float32),
            jax.ShapeDtypeStruct((N, 128), jnp.bfloat16),
        ],
    )(Q, Wcat, bcat)


_BM = 128
_NB = N // _BM


def _stage_kernel(a_ref, x_hbm, y_ref, z_ref, abf_ref, xbf_ref, sem):
    i = pl.program_id(0)

    @pl.when(i == 0)
    def _():
        cp = pltpu.make_async_copy(x_hbm, xbf_ref, sem)
        cp.start()
        cp.wait()

    @pl.when(i < _NB)
    def _():
        ab = a_ref[...].astype(jnp.bfloat16)
        abf_ref[pl.ds(i * _BM, _BM), :] = ab
        y_ref[pl.ds(i * _BM, _BM), :] = jnp.dot(
            ab, xbf_ref[...], preferred_element_type=jnp.float32)

    @pl.when(i == _NB)
    def _():
        t = y_ref[:, 64:128].astype(jnp.bfloat16)
        z_ref[...] = jnp.dot(abf_ref[...], t, preferred_element_type=jnp.float32)


def _stage(A, Xbf):
    return pl.pallas_call(
        _stage_kernel,
        grid=(_NB + 1,),
        in_specs=[
            pl.BlockSpec((_BM, N), lambda i: (jnp.minimum(i, _NB - 1), 0)),
            pl.BlockSpec(memory_space=pl.ANY),
        ],
        out_specs=[
            pl.BlockSpec((N, 128), lambda i: (0, 0)),
            pl.BlockSpec((N, 64), lambda i: (0, 0)),
        ],
        out_shape=[
            jax.ShapeDtypeStruct((N, 128), jnp.float32),
            jax.ShapeDtypeStruct((N, 64), jnp.float32),
        ],
        scratch_shapes=[
            pltpu.VMEM((N, N), jnp.bfloat16),
            pltpu.VMEM((N, 128), jnp.bfloat16),
            pltpu.SemaphoreType.DMA,
        ],
    )(A, Xbf)


def _amp_kernel(f1_ref, f2_ref, o_ref):
    f1 = f1_ref[...]
    f2 = f2_ref[...]
    n1 = jnp.maximum(jnp.sqrt(jnp.sum(f1 * f1, axis=0)), 1e-8)
    n2 = jnp.maximum(jnp.sqrt(jnp.sum(f2 * f2, axis=0)), 1e-8)
    cs = jnp.sum(f1 * f2, axis=0) / (n1 * n2)
    o_ref[...] = jax.nn.sigmoid(1.0 - cs)[None, :]


def _amp(f11, f21):
    return pl.pallas_call(
        _amp_kernel,
        out_shape=jax.ShapeDtypeStruct((1, 192), jnp.float32),
    )(f11, f21)


def _fg_kernel(f_ref, amp_ref, w_ref, g64_ref, gh_ref):
    acc = jnp.dot(f_ref[...] * amp_ref[...], w_ref[...],
                  preferred_element_type=jnp.float32)
    g64_ref[...] = acc[:, :64]
    gh_ref[...] = acc[:, 64:].astype(jnp.bfloat16)


def _fg(f, amp, W2cat):
    BM = 512
    return pl.pallas_call(
        _fg_kernel,
        grid=(N // BM,),
        in_specs=[
            pl.BlockSpec((BM, 192), lambda i: (i, 0)),
            pl.BlockSpec((1, 192), lambda i: (0, 0)),
            pl.BlockSpec((192, 192), lambda i: (0, 0)),
        ],
        out_specs=[
            pl.BlockSpec((BM, 64), lambda i: (i, 0)),
            pl.BlockSpec((BM, 128), lambda i: (i, 0)),
        ],
        out_shape=[
            jax.ShapeDtypeStruct((N, 64), jnp.float32),
            jax.ShapeDtypeStruct((N, 128), jnp.bfloat16),
        ],
    )(f, amp, W2cat)


def _mm_kernel(x_ref, w_ref, o_ref):
    o_ref[...] = jnp.dot(x_ref[...], w_ref[...], preferred_element_type=jnp.float32)


def _feats_project(feats0, Wp):
    BM = 512
    return pl.pallas_call(
        _mm_kernel,
        grid=(N // BM,),
        in_specs=[
            pl.BlockSpec((BM, 384), lambda i: (i, 0)),
            pl.BlockSpec((384, 32), lambda i: (0, 0)),
        ],
        out_specs=pl.BlockSpec((BM, 32), lambda i: (i, 0)),
        out_shape=jax.ShapeDtypeStruct((N, 32), jnp.float32),
    )(feats0, Wp)


def _s_kernel(s_ref, f_hbm, c_hbm, o_ref, fs_ref, cs_ref, sem1, sem2):
    i = pl.program_id(0)

    @pl.when(i == 0)
    def _():
        cp1 = pltpu.make_async_copy(f_hbm, fs_ref, sem1)
        cp1.start()
        cp2 = pltpu.make_async_copy(c_hbm, cs_ref, sem2)
        cp2.start()
        cp1.wait()
        cp2.wait()

    y = jnp.dot(s_ref[...], fs_ref[...], preferred_element_type=jnp.float32)
    y = y + cs_ref[...]
    o_ref[...] = jnp.where(y >= 0, y, 0.01 * y)


def _s_matmul(S, F2, cp):
    BM = 512
    return pl.pallas_call(
        _s_kernel,
        grid=(NPIX // BM,),
        in_specs=[
            pl.BlockSpec((BM, N), lambda i: (i, 0)),
            pl.BlockSpec(memory_space=pl.ANY),
            pl.BlockSpec(memory_space=pl.ANY),
        ],
        out_specs=pl.BlockSpec((BM, 32), lambda i: (i, 0)),
        out_shape=jax.ShapeDtypeStruct((NPIX, 32), jnp.float32),
        scratch_shapes=[
            pltpu.VMEM((N, 32), jnp.float32),
            pltpu.VMEM((1, 32), jnp.float32),
            pltpu.SemaphoreType.DMA,
            pltpu.SemaphoreType.DMA,
        ],
    )(S, F2, cp)


def _head_kernel(x_ref, dwk_ref, dwb_ref, fcw_ref, fcb_ref, o_ref):
    x = x_ref[...]
    zpad = jnp.zeros((129, 32), jnp.float32)
    xp = jnp.concatenate([zpad, x, zpad], axis=0)
    wcol = jax.lax.broadcasted_iota(jnp.int32, (NPIX, 1), 0) % WW
    acc = jnp.zeros((NPIX, 32), jnp.float32)
    k = 0
    for dy in (-1, 0, 1):
        for dx in (-1, 0, 1):
            s = dy * WW + dx
            sh = jax.lax.slice(xp, (129 + s, 0), (129 + s + NPIX, 32))
            if dx == -1:
                sh = jnp.where(wcol >= 1, sh, 0.0)
            elif dx == 1:
                sh = jnp.where(wcol <= WW - 2, sh, 0.0)
            acc = acc + sh * dwk_ref[k, :][None, :]
            k += 1
    y = acc + dwb_ref[...]
    y = jnp.where(y >= 0, y, 0.01 * y)
    logits = jnp.dot(y, fcw_ref[...], preferred_element_type=jnp.float32)
    logits = logits + fcb_ref[...]
    m = jnp.max(logits, axis=1, keepdims=True)
    e = jnp.exp(logits - m)
    o_ref[...] = e / jnp.sum(e, axis=1, keepdims=True)


def _head(X1, dwk, dwb, fcw, fcb):
    return pl.pallas_call(
        _head_kernel,
        out_shape=jax.ShapeDtypeStruct((NPIX, 16), jnp.float32),
    )(X1, dwk, dwb, fcw, fcb)


def kernel(A1, Q1, A2, Q2, S, W1, b1, W2, b2, bn_gamma, bn_beta, bn_mean,
           bn_var, pw_w, dw_w, dw_b, fc_w, fc_b):
    Wcat = jnp.concatenate([W1[0], W1[1], W1[2]], axis=1)
    bcat = jnp.reshape(b1, (1, 192))
    W2cat = jnp.concatenate([W2[0], W2[1], W2[2]], axis=1)

    def branch_sparse(A, Q):
        s0, H = _input_transform(Q, Wcat, bcat)
        Y1, s2 = _stage(A, H)
        return jnp.concatenate([s0, Y1[:, :64], s2], axis=1)

    f11 = branch_sparse(A1, Q1)
    f21 = branch_sparse(A2, Q2)
    amp = _amp(f11, f21)

    def branch_dense(A, f):
        d0, Gh = _fg(f, amp, W2cat)
        Y3, d2 = _stage(A, Gh)
        return jnp.concatenate([d0, Y3[:, :64], d2], axis=1)

    f12 = branch_dense(A1, f11)
    f22 = branch_dense(A2, f21)
    feats0 = jnp.concatenate([f12, f22], axis=1)

    scale = bn_gamma / jnp.sqrt(bn_var + 1e-5)
    shift = bn_beta - bn_mean * scale
    pwT = pw_w[:, :, 0, 0].T
    Wp = scale[:, None] * pwT
    bvec = jnp.concatenate([jnp.reshape(b2, (192,))] * 2)[None, :]
    cp_total = bvec @ Wp + shift[None, :] @ pwT

    F2 = _feats_project(feats0, Wp)
    X1 = _s_matmul(S, F2, cp_total)

    dwk = jnp.transpose(dw_w[:, 0], (1, 2, 0)).reshape(9, 32)
    return _head(X1, dwk, dw_b[None, :], fc_w, fc_b[None, :])


# speedup vs baseline: 1.3037x; 1.1061x over previous
import jax
import jax.numpy as jnp
from jax.experimental import pallas as pl
from jax.experimental.pallas import tpu as pltpu

N = 4096
F = 128
HH = 128
WW = 128
NPIX = HH * WW


def _input_kernel(x_ref, w_ref, b_ref, s0_ref, h_ref):
    acc = jnp.dot(x_ref[...], w_ref[...], preferred_element_type=jnp.float32)
    acc = jnp.maximum(acc + b_ref[...], 0.0)
    s0_ref[...] = acc[:, :64]
    h_ref[...] = acc[:, 64:].astype(jnp.bfloat16)


def _input_transform(Q, Wcat, bcat):
    BM = 512
    return pl.pallas_call(
        _input_kernel,
        grid=(N // BM,),
        in_specs=[
            pl.BlockSpec((BM, F), lambda i: (i, 0)),
            pl.BlockSpec((F, 192), lambda i: (0, 0)),
            pl.BlockSpec((1, 192), lambda i: (0, 0)),
        ],
        out_specs=[
            pl.BlockSpec((BM, 64), lambda i: (i, 0)),
            pl.BlockSpec((BM, 128), lambda i: (i, 0)),
        ],
        out_shape=[
            jax.ShapeDtypeStruct((N, 64), jnp.float32),
            jax.ShapeDtypeStruct((N, 128), jnp.bfloat16),
        ],
    )(Q, Wcat, bcat)


_BM = 256
_NB = N // _BM


def _stage_kernel(a_ref, x_hbm, y_ref, z_ref, abf_ref, xbf_ref, sem):
    i = pl.program_id(0)

    @pl.when(i == 0)
    def _():
        cp = pltpu.make_async_copy(x_hbm, xbf_ref, sem)
        cp.start()
        cp.wait()

    @pl.when(i < _NB)
    def _():
        ab = a_ref[...].astype(jnp.bfloat16)
        abf_ref[pl.ds(i * _BM, _BM), :] = ab
        y_ref[pl.ds(i * _BM, _BM), :] = jnp.dot(
            ab, xbf_ref[...], preferred_element_type=jnp.float32)

    @pl.when(i == _NB)
    def _():
        t = y_ref[:, 64:128].astype(jnp.bfloat16)
        z_ref[...] = jnp.dot(abf_ref[...], t, preferred_element_type=jnp.float32)


def _stage(A, Xbf):
    return pl.pallas_call(
        _stage_kernel,
        grid=(_NB + 1,),
        in_specs=[
            pl.BlockSpec((_BM, N), lambda i: (jnp.minimum(i, _NB - 1), 0)),
            pl.BlockSpec(memory_space=pl.ANY),
        ],
        out_specs=[
            pl.BlockSpec((N, 128), lambda i: (0, 0)),
            pl.BlockSpec((N, 64), lambda i: (0, 0)),
        ],
        out_shape=[
            jax.ShapeDtypeStruct((N, 128), jnp.float32),
            jax.ShapeDtypeStruct((N, 64), jnp.float32),
        ],
        scratch_shapes=[
            pltpu.VMEM((N, N), jnp.bfloat16),
            pltpu.VMEM((N, 128), jnp.bfloat16),
            pltpu.SemaphoreType.DMA,
        ],
    )(A, Xbf)


def _amp_kernel(f1_ref, f2_ref, o_ref):
    f1 = f1_ref[...]
    f2 = f2_ref[...]
    n1 = jnp.maximum(jnp.sqrt(jnp.sum(f1 * f1, axis=0)), 1e-8)
    n2 = jnp.maximum(jnp.sqrt(jnp.sum(f2 * f2, axis=0)), 1e-8)
    cs = jnp.sum(f1 * f2, axis=0) / (n1 * n2)
    o_ref[...] = jax.nn.sigmoid(1.0 - cs)[None, :]


def _amp(f11, f21):
    return pl.pallas_call(
        _amp_kernel,
        out_shape=jax.ShapeDtypeStruct((1, 192), jnp.float32),
    )(f11, f21)


def _fg_kernel(f_ref, amp_ref, w_ref, g64_ref, gh_ref):
    acc = jnp.dot(f_ref[...] * amp_ref[...], w_ref[...],
                  preferred_element_type=jnp.float32)
    g64_ref[...] = acc[:, :64]
    gh_ref[...] = acc[:, 64:].astype(jnp.bfloat16)


def _fg(f, amp, W2cat):
    BM = 512
    return pl.pallas_call(
        _fg_kernel,
        grid=(N // BM,),
        in_specs=[
            pl.BlockSpec((BM, 192), lambda i: (i, 0)),
            pl.BlockSpec((1, 192), lambda i: (0, 0)),
            pl.BlockSpec((192, 192), lambda i: (0, 0)),
        ],
        out_specs=[
            pl.BlockSpec((BM, 64), lambda i: (i, 0)),
            pl.BlockSpec((BM, 128), lambda i: (i, 0)),
        ],
        out_shape=[
            jax.ShapeDtypeStruct((N, 64), jnp.float32),
            jax.ShapeDtypeStruct((N, 128), jnp.bfloat16),
        ],
    )(f, amp, W2cat)


def _mm_kernel(x_ref, w_ref, o_ref):
    o_ref[...] = jnp.dot(x_ref[...], w_ref[...], preferred_element_type=jnp.float32)


def _feats_project(feats0, Wp):
    BM = 512
    return pl.pallas_call(
        _mm_kernel,
        grid=(N // BM,),
        in_specs=[
            pl.BlockSpec((BM, 384), lambda i: (i, 0)),
            pl.BlockSpec((384, 32), lambda i: (0, 0)),
        ],
        out_specs=pl.BlockSpec((BM, 32), lambda i: (i, 0)),
        out_shape=jax.ShapeDtypeStruct((N, 32), jnp.float32),
    )(feats0, Wp)


def _s_kernel(s_ref, f_hbm, c_hbm, o_ref, fs_ref, cs_ref, sem1, sem2):
    i = pl.program_id(0)

    @pl.when(i == 0)
    def _():
        cp1 = pltpu.make_async_copy(f_hbm, fs_ref, sem1)
        cp1.start()
        cp2 = pltpu.make_async_copy(c_hbm, cs_ref, sem2)
        cp2.start()
        cp1.wait()
        cp2.wait()

    y = jnp.dot(s_ref[...], fs_ref[...], preferred_element_type=jnp.float32)
    y = y + cs_ref[...]
    o_ref[...] = jnp.where(y >= 0, y, 0.01 * y)


def _s_matmul(S, F2, cp):
    BM = 512
    return pl.pallas_call(
        _s_kernel,
        grid=(NPIX // BM,),
        in_specs=[
            pl.BlockSpec((BM, N), lambda i: (i, 0)),
            pl.BlockSpec(memory_space=pl.ANY),
            pl.BlockSpec(memory_space=pl.ANY),
        ],
        out_specs=pl.BlockSpec((BM, 32), lambda i: (i, 0)),
        out_shape=jax.ShapeDtypeStruct((NPIX, 32), jnp.float32),
        scratch_shapes=[
            pltpu.VMEM((N, 32), jnp.float32),
            pltpu.VMEM((1, 32), jnp.float32),
            pltpu.SemaphoreType.DMA,
            pltpu.SemaphoreType.DMA,
        ],
    )(S, F2, cp)


def _head_kernel(x_ref, dwk_ref, dwb_ref, fcw_ref, fcb_ref, o_ref):
    x = x_ref[...]
    zpad = jnp.zeros((129, 32), jnp.float32)
    xp = jnp.concatenate([zpad, x, zpad], axis=0)
    wcol = jax.lax.broadcasted_iota(jnp.int32, (NPIX, 1), 0) % WW
    acc = jnp.zeros((NPIX, 32), jnp.float32)
    k = 0
    for dy in (-1, 0, 1):
        for dx in (-1, 0, 1):
            s = dy * WW + dx
            sh = jax.lax.slice(xp, (129 + s, 0), (129 + s + NPIX, 32))
            if dx == -1:
                sh = jnp.where(wcol >= 1, sh, 0.0)
            elif dx == 1:
                sh = jnp.where(wcol <= WW - 2, sh, 0.0)
            acc = acc + sh * dwk_ref[k, :][None, :]
            k += 1
    y = acc + dwb_ref[...]
    y = jnp.where(y >= 0, y, 0.01 * y)
    logits = jnp.dot(y, fcw_ref[...], preferred_element_type=jnp.float32)
    logits = logits + fcb_ref[...]
    m = jnp.max(logits, axis=1, keepdims=True)
    e = jnp.exp(logits - m)
    o_ref[...] = e / jnp.sum(e, axis=1, keepdims=True)


def _head(X1, dwk, dwb, fcw, fcb):
    return pl.pallas_call(
        _head_kernel,
        out_shape=jax.ShapeDtypeStruct((NPIX, 16), jnp.float32),
    )(X1, dwk, dwb, fcw, fcb)


def kernel(A1, Q1, A2, Q2, S, W1, b1, W2, b2, bn_gamma, bn_beta, bn_mean,
           bn_var, pw_w, dw_w, dw_b, fc_w, fc_b):
    Wcat = jnp.concatenate([W1[0], W1[1], W1[2]], axis=1)
    bcat = jnp.reshape(b1, (1, 192))
    W2cat = jnp.concatenate([W2[0], W2[1], W2[2]], axis=1)

    def branch_sparse(A, Q):
        s0, H = _input_transform(Q, Wcat, bcat)
        Y1, s2 = _stage(A, H)
        return jnp.concatenate([s0, Y1[:, :64], s2], axis=1)

    f11 = branch_sparse(A1, Q1)
    f21 = branch_sparse(A2, Q2)
    amp = _amp(f11, f21)

    def branch_dense(A, f):
        d0, Gh = _fg(f, amp, W2cat)
        Y3, d2 = _stage(A, Gh)
        return jnp.concatenate([d0, Y3[:, :64], d2], axis=1)

    f12 = branch_dense(A1, f11)
    f22 = branch_dense(A2, f21)
    feats0 = jnp.concatenate([f12, f22], axis=1)

    scale = bn_gamma / jnp.sqrt(bn_var + 1e-5)
    shift = bn_beta - bn_mean * scale
    pwT = pw_w[:, :, 0, 0].T
    Wp = scale[:, None] * pwT
    bvec = jnp.concatenate([jnp.reshape(b2, (192,))] * 2)[None, :]
    cp_total = bvec @ Wp + shift[None, :] @ pwT

    F2 = _feats_project(feats0, Wp)
    X1 = _s_matmul(S, F2, cp_total)

    dwk = jnp.transpose(dw_w[:, 0], (1, 2, 0)).reshape(9, 32)
    return _head(X1, dwk, dw_b[None, :], fc_w, fc_b[None, :])
